# baseline (speedup 1.0000x reference)
import functools

import jax
import jax.numpy as jnp
from jax import lax
from jax.experimental import pallas as pl
from jax.experimental.pallas import tpu as pltpu
from jax.experimental.pallas import tpu_sc as plsc

N_NODES = 10000
D = 128
EPS = 1e-5

NC = 2
NS = 16
NW = NC * NS
CHUNK = 80
DEG_W = 128
N_PAD = 10240
ROWS_PT = N_PAD // NS

_MESH = dict(core_axis_name="c", subcore_axis_name="s", num_cores=NC,
             num_subcores=NS)


NBUF = 4
IBLK = 8
LA = NBUF - 1


@functools.lru_cache(maxsize=None)
def _deg_kernel(nch):
  @functools.partial(
      pl.kernel,
      out_type=jax.ShapeDtypeStruct((NC, N_PAD, DEG_W), jnp.float32),
      mesh=plsc.VectorSubcoreMesh(**_MESH),
      scratch_types=[
          pltpu.VMEM((nch, CHUNK), jnp.int32),
          pltpu.VMEM((CHUNK, DEG_W), jnp.float32),
          pltpu.VMEM_SHARED((N_PAD, DEG_W), jnp.float32),
          pltpu.SemaphoreType.DMA,
      ],
  )
  def deg(dst_hbm, ones_hbm, zeros_hbm, out_hbm, didx_v, ones_v, acc_sh, sem):
    cid = lax.axis_index("c")
    sid = lax.axis_index("s")
    wid = cid * NS + sid
    pltpu.sync_copy(zeros_hbm, acc_sh.at[pl.ds(sid * ROWS_PT, ROWS_PT)])
    pltpu.sync_copy(ones_hbm, ones_v)
    pltpu.sync_copy(dst_hbm.at[wid], didx_v)
    plsc.subcore_barrier()

    def fire(c, carry):
      pltpu.async_copy(ones_v, acc_sh.at[didx_v.at[c]], sem, add=True)
      return carry

    lax.fori_loop(0, nch, fire, 0)

    def drain(c, carry):
      pltpu.make_async_copy(ones_v, acc_sh.at[didx_v.at[c]], sem).wait()
      return carry

    lax.fori_loop(0, nch, drain, 0)
    plsc.subcore_barrier()
    pltpu.sync_copy(acc_sh.at[pl.ds(sid * ROWS_PT, ROWS_PT)],
                    out_hbm.at[cid, pl.ds(sid * ROWS_PT, ROWS_PT)])

  return deg


@functools.lru_cache(maxsize=None)
def _scatter_kernel(nch):
  nblk = nch // IBLK
  assert nch % NBUF == 0 and IBLK % NBUF == 0

  @functools.partial(
      pl.kernel,
      out_type=jax.ShapeDtypeStruct((NC, N_PAD, D), jnp.float32),
      mesh=plsc.VectorSubcoreMesh(**_MESH),
      scratch_types=[
          pltpu.VMEM((2, IBLK, CHUNK), jnp.int32),
          pltpu.VMEM((2, IBLK, CHUNK), jnp.int32),
          pltpu.VMEM((NBUF, CHUNK, D), jnp.float32),
          pltpu.VMEM_SHARED((N_PAD, D), jnp.float32),
          pltpu.SemaphoreType.DMA,
          pltpu.SemaphoreType.DMA((NBUF,)),
          pltpu.SemaphoreType.DMA((NBUF,)),
      ],
  )
  def scat(g_hbm, src_hbm, dst_hbm, zeros_hbm, out_hbm,
           sidx_v, didx_v, rows_v, acc_sh, isem, gsem, ssem):
    cid = lax.axis_index("c")
    sid = lax.axis_index("s")
    wid = cid * NS + sid
    pltpu.sync_copy(zeros_hbm, acc_sh.at[pl.ds(sid * ROWS_PT, ROWS_PT)])
    pltpu.sync_copy(src_hbm.at[wid, 0], sidx_v.at[0])
    pltpu.sync_copy(dst_hbm.at[wid, 0], didx_v.at[0])
    for b in range(LA):
      pltpu.async_copy(g_hbm.at[sidx_v.at[0, b]], rows_v.at[b], gsem.at[b])
    plsc.subcore_barrier()

    def blk_body(blk, carry):
      p = lax.rem(blk, 2)
      pn = 1 - p
      for k in range(IBLK):
        q = k % NBUF
        qp = (k - 1) % NBUF
        pltpu.make_async_copy(g_hbm.at[sidx_v.at[p, k]], rows_v.at[q],
                              gsem.at[q]).wait()
        pltpu.async_copy(rows_v.at[q], acc_sh.at[didx_v.at[p, k]],
                         ssem.at[q], add=True)
        if k == 0:
          @pl.when(blk > 0)
          def _():
            pltpu.make_async_copy(rows_v.at[qp], acc_sh.at[didx_v.at[p, 0]],
                                  ssem.at[qp]).wait()

          @pl.when(blk + 1 < nblk)
          def _():
            pltpu.async_copy(src_hbm.at[wid, blk + 1], sidx_v.at[pn], isem)
            pltpu.async_copy(dst_hbm.at[wid, blk + 1], didx_v.at[pn], isem)
        else:
          pltpu.make_async_copy(rows_v.at[qp], acc_sh.at[didx_v.at[p, k]],
                                ssem.at[qp]).wait()
        if k < IBLK - LA:
          pltpu.async_copy(g_hbm.at[sidx_v.at[p, k + LA]], rows_v.at[qp],
                           gsem.at[qp])
        else:
          kk = k + LA - IBLK

          if kk == 0:
            @pl.when(blk + 1 < nblk)
            def _():
              pltpu.make_async_copy(src_hbm.at[wid, blk + 1], sidx_v.at[pn],
                                    isem).wait()
              pltpu.make_async_copy(dst_hbm.at[wid, blk + 1], didx_v.at[pn],
                                    isem).wait()

          @pl.when(blk + 1 < nblk)
          def _():
            pltpu.async_copy(g_hbm.at[sidx_v.at[pn, kk]], rows_v.at[qp],
                             gsem.at[qp])

      return carry

    lax.fori_loop(0, nblk, blk_body, 0)
    q_last = (IBLK - 1) % NBUF
    p_last = (nblk - 1) % 2
    pltpu.make_async_copy(rows_v.at[q_last],
                          acc_sh.at[didx_v.at[p_last, IBLK - 1]],
                          ssem.at[q_last]).wait()
    plsc.subcore_barrier()
    pltpu.sync_copy(acc_sh.at[pl.ds(sid * ROWS_PT, ROWS_PT)],
                    out_hbm.at[cid, pl.ds(sid * ROWS_PT, ROWS_PT)])

  return scat


BR = 512
BR3 = 400


def _mm_body(x_ref, w_ref, u_ref):
  u_ref[...] = jnp.dot(x_ref[...], w_ref[...],
                       preferred_element_type=jnp.float32)


def _b1_body(u_ref, d0_ref, d1_ref, g_ref, dinv_ref):
  deg = d0_ref[0][:, 0:1] + d1_ref[0][:, 0:1] + 1.0
  dinv = lax.rsqrt(deg)
  g_ref[...] = u_ref[...] * dinv
  dinv_ref[...] = jnp.broadcast_to(dinv, dinv_ref.shape)


def _post_block(p0, p1, g_ref, dinv, b_ref, lw_ref, lb_ref, a_ref):
  h = (p0 + p1 + g_ref[...]) * dinv + b_ref[...]
  h = jnp.where(h >= 0, h, a_ref[...] * h)
  mu = jnp.mean(h, axis=-1, keepdims=True)
  var = jnp.mean((h - mu) ** 2, axis=-1, keepdims=True)
  return (h - mu) * lax.rsqrt(var + EPS) * lw_ref[...] + lb_ref[...]


def _b2_body(p0_ref, p1_ref, g1_ref, dv_ref, b_ref, lw_ref, lb_ref,
             a_ref, w_ref, out_ref):
  dinv = dv_ref[...][:, 0:1]
  h = _post_block(p0_ref[0], p1_ref[0], g1_ref, dinv, b_ref, lw_ref, lb_ref,
                  a_ref)
  out_ref[...] = jnp.dot(h, w_ref[...],
                         preferred_element_type=jnp.float32) * dinv


def _b3_body(p0_ref, p1_ref, g2_ref, dv_ref, b_ref, lw_ref, lb_ref,
             a_ref, out_ref):
  dinv = dv_ref[...][:, 0:1]
  out_ref[...] = _post_block(p0_ref[0], p1_ref[0], g2_ref, dinv, b_ref,
                             lw_ref, lb_ref, a_ref)


def _row_spec():
  return pl.BlockSpec((BR, D), lambda i: (i, 0))


def _part_spec(c):
  return pl.BlockSpec((1, BR, D), lambda i, c=c: (c, i, 0))


def _degp_spec(c):
  return pl.BlockSpec((1, BR, DEG_W), lambda i, c=c: (c, i, 0))


def _dinv_spec():
  return pl.BlockSpec((BR, 8), lambda i: (i, 0))


def _vec_spec():
  return pl.BlockSpec((1, D), lambda i: (0, 0))


def _full_spec():
  return pl.BlockSpec((D, D), lambda i: (0, 0))


_GRID = (N_PAD // BR,)
_ROWS_OUT = jax.ShapeDtypeStruct((N_PAD, D), jnp.float32)


def kernel(x, edge_index, W1, b1, W2, b2, prelu_a, ln_w, ln_b):
  e = edge_index.shape[1]
  ept = -(-e // (NW * CHUNK * IBLK)) * CHUNK * IBLK
  e_pad = ept * NW
  nch = ept // CHUNK
  pad = N_NODES + jnp.arange(e_pad - e, dtype=jnp.int32) % (N_PAD - N_NODES)
  nblk = nch // IBLK
  src3 = jnp.concatenate([edge_index[0].astype(jnp.int32), pad])
  src3 = src3.reshape(NW, nblk, IBLK, CHUNK)
  dstf = jnp.concatenate([edge_index[1].astype(jnp.int32), pad])
  dst3 = dstf.reshape(NW, nblk, IBLK, CHUNK)
  dst = dstf.reshape(NW, nch, CHUNK)
  x_pad = jnp.pad(x, ((0, N_PAD - N_NODES), (0, 0)))

  ones_c = jnp.ones((CHUNK, DEG_W), jnp.float32)
  zeros_d = jnp.zeros((ROWS_PT, DEG_W), jnp.float32)
  zeros_r = jnp.zeros((ROWS_PT, D), jnp.float32)

  b1v = b1.reshape(1, D)
  b2v = b2.reshape(1, D)
  lwv = ln_w.reshape(1, D)
  lbv = ln_b.reshape(1, D)
  av = jnp.broadcast_to(prelu_a.reshape(1, 1), (1, D))

  degp = _deg_kernel(nch)(dst, ones_c, zeros_d)

  u1 = pl.pallas_call(
      _mm_body,
      grid=_GRID,
      in_specs=[_row_spec(), _full_spec()],
      out_specs=_row_spec(),
      out_shape=_ROWS_OUT,
  )(x_pad, W1)

  g1, dinv = pl.pallas_call(
      _b1_body,
      grid=_GRID,
      in_specs=[_row_spec(), _degp_spec(0), _degp_spec(1)],
      out_specs=[_row_spec(), _dinv_spec()],
      out_shape=[_ROWS_OUT, jax.ShapeDtypeStruct((N_PAD, 8), jnp.float32)],
  )(u1, degp, degp)

  p = _scatter_kernel(nch)(g1, src3, dst3, zeros_r)

  g2 = pl.pallas_call(
      _b2_body,
      grid=_GRID,
      in_specs=[_part_spec(0), _part_spec(1), _row_spec(), _dinv_spec(),
                _vec_spec(), _vec_spec(), _vec_spec(), _vec_spec(),
                _full_spec()],
      out_specs=_row_spec(),
      out_shape=_ROWS_OUT,
  )(p, p, g1, dinv, b1v, lwv, lbv, av, W2)

  p2 = _scatter_kernel(nch)(g2, src3, dst3, zeros_r)

  out = pl.pallas_call(
      _b3_body,
      grid=(N_NODES // BR3,),
      in_specs=[
          pl.BlockSpec((1, BR3, D), lambda i: (0, i, 0)),
          pl.BlockSpec((1, BR3, D), lambda i: (1, i, 0)),
          pl.BlockSpec((BR3, D), lambda i: (i, 0)),
          pl.BlockSpec((BR3, 8), lambda i: (i, 0)),
          _vec_spec(), _vec_spec(), _vec_spec(), _vec_spec(),
      ],
      out_specs=pl.BlockSpec((BR3, D), lambda i: (i, 0)),
      out_shape=jax.ShapeDtypeStruct((N_NODES, D), jnp.float32),
  )(p2, p2, g2, dinv, b2v, lwv, lbv, av)

  return out


# speedup vs baseline: 1.0353x; 1.0353x over previous
import functools

import jax
import jax.numpy as jnp
from jax import lax
from jax.experimental import pallas as pl
from jax.experimental.pallas import tpu as pltpu
from jax.experimental.pallas import tpu_sc as plsc

N_NODES = 10000
D = 128
EPS = 1e-5

NC = 2
NS = 16
NW = NC * NS
CHUNK = 112
DEG_W = 128
N_PAD = 10240
ROWS_PT = N_PAD // NS

_MESH = dict(core_axis_name="c", subcore_axis_name="s", num_cores=NC,
             num_subcores=NS)


NBUF = 3
IBLK = 6
LA = NBUF - 1


@functools.lru_cache(maxsize=None)
def _deg_kernel(nch):
  @functools.partial(
      pl.kernel,
      out_type=jax.ShapeDtypeStruct((NC, N_PAD, DEG_W), jnp.float32),
      mesh=plsc.VectorSubcoreMesh(**_MESH),
      scratch_types=[
          pltpu.VMEM((nch, CHUNK), jnp.int32),
          pltpu.VMEM((CHUNK, DEG_W), jnp.float32),
          pltpu.VMEM_SHARED((N_PAD, DEG_W), jnp.float32),
          pltpu.SemaphoreType.DMA,
      ],
  )
  def deg(dst_hbm, ones_hbm, zeros_hbm, out_hbm, didx_v, ones_v, acc_sh, sem):
    cid = lax.axis_index("c")
    sid = lax.axis_index("s")
    wid = cid * NS + sid
    pltpu.sync_copy(zeros_hbm, acc_sh.at[pl.ds(sid * ROWS_PT, ROWS_PT)])
    pltpu.sync_copy(ones_hbm, ones_v)
    pltpu.sync_copy(dst_hbm.at[wid], didx_v)
    plsc.subcore_barrier()

    def fire(c, carry):
      pltpu.async_copy(ones_v, acc_sh.at[didx_v.at[c]], sem, add=True)
      return carry

    lax.fori_loop(0, nch, fire, 0)

    def drain(c, carry):
      pltpu.make_async_copy(ones_v, acc_sh.at[didx_v.at[c]], sem).wait()
      return carry

    lax.fori_loop(0, nch, drain, 0)
    plsc.subcore_barrier()
    pltpu.sync_copy(acc_sh.at[pl.ds(sid * ROWS_PT, ROWS_PT)],
                    out_hbm.at[cid, pl.ds(sid * ROWS_PT, ROWS_PT)])

  return deg


@functools.lru_cache(maxsize=None)
def _scatter_kernel(nch):
  nblk = nch // IBLK
  assert nch % NBUF == 0 and IBLK % NBUF == 0

  @functools.partial(
      pl.kernel,
      out_type=jax.ShapeDtypeStruct((NC, N_PAD, D), jnp.float32),
      mesh=plsc.VectorSubcoreMesh(**_MESH),
      scratch_types=[
          pltpu.VMEM((2, IBLK, CHUNK), jnp.int32),
          pltpu.VMEM((2, IBLK, CHUNK), jnp.int32),
          pltpu.VMEM((NBUF, CHUNK, D), jnp.float32),
          pltpu.VMEM_SHARED((N_PAD, D), jnp.float32),
          pltpu.SemaphoreType.DMA,
          pltpu.SemaphoreType.DMA((NBUF,)),
          pltpu.SemaphoreType.DMA((NBUF,)),
      ],
  )
  def scat(g_hbm, src_hbm, dst_hbm, zeros_hbm, out_hbm,
           sidx_v, didx_v, rows_v, acc_sh, isem, gsem, ssem):
    cid = lax.axis_index("c")
    sid = lax.axis_index("s")
    wid = cid * NS + sid
    pltpu.sync_copy(zeros_hbm, acc_sh.at[pl.ds(sid * ROWS_PT, ROWS_PT)])
    pltpu.sync_copy(src_hbm.at[wid, 0], sidx_v.at[0])
    pltpu.sync_copy(dst_hbm.at[wid, 0], didx_v.at[0])
    for b in range(LA):
      pltpu.async_copy(g_hbm.at[sidx_v.at[0, b]], rows_v.at[b], gsem.at[b])
    plsc.subcore_barrier()

    def blk_body(blk, carry):
      p = lax.rem(blk, 2)
      pn = 1 - p
      for k in range(IBLK):
        q = k % NBUF
        qp = (k - 1) % NBUF
        pltpu.make_async_copy(g_hbm.at[sidx_v.at[p, k]], rows_v.at[q],
                              gsem.at[q]).wait()
        pltpu.async_copy(rows_v.at[q], acc_sh.at[didx_v.at[p, k]],
                         ssem.at[q], add=True)
        if k == 0:
          @pl.when(blk > 0)
          def _():
            pltpu.make_async_copy(rows_v.at[qp], acc_sh.at[didx_v.at[p, 0]],
                                  ssem.at[qp]).wait()

          @pl.when(blk + 1 < nblk)
          def _():
            pltpu.async_copy(src_hbm.at[wid, blk + 1], sidx_v.at[pn], isem)
            pltpu.async_copy(dst_hbm.at[wid, blk + 1], didx_v.at[pn], isem)
        else:
          pltpu.make_async_copy(rows_v.at[qp], acc_sh.at[didx_v.at[p, k]],
                                ssem.at[qp]).wait()
        if k < IBLK - LA:
          pltpu.async_copy(g_hbm.at[sidx_v.at[p, k + LA]], rows_v.at[qp],
                           gsem.at[qp])
        else:
          kk = k + LA - IBLK

          if kk == 0:
            @pl.when(blk + 1 < nblk)
            def _():
              pltpu.make_async_copy(src_hbm.at[wid, blk + 1], sidx_v.at[pn],
                                    isem).wait()
              pltpu.make_async_copy(dst_hbm.at[wid, blk + 1], didx_v.at[pn],
                                    isem).wait()

          @pl.when(blk + 1 < nblk)
          def _():
            pltpu.async_copy(g_hbm.at[sidx_v.at[pn, kk]], rows_v.at[qp],
                             gsem.at[qp])

      return carry

    lax.fori_loop(0, nblk, blk_body, 0)
    q_last = (IBLK - 1) % NBUF
    p_last = (nblk - 1) % 2
    pltpu.make_async_copy(rows_v.at[q_last],
                          acc_sh.at[didx_v.at[p_last, IBLK - 1]],
                          ssem.at[q_last]).wait()
    plsc.subcore_barrier()
    pltpu.sync_copy(acc_sh.at[pl.ds(sid * ROWS_PT, ROWS_PT)],
                    out_hbm.at[cid, pl.ds(sid * ROWS_PT, ROWS_PT)])

  return scat


BR = 512
BR3 = 400


def _mm_body(x_ref, w_ref, u_ref):
  u_ref[...] = jnp.dot(x_ref[...], w_ref[...],
                       preferred_element_type=jnp.float32)


def _b1_body(u_ref, d0_ref, d1_ref, g_ref, dinv_ref):
  deg = d0_ref[0][:, 0:1] + d1_ref[0][:, 0:1] + 1.0
  dinv = lax.rsqrt(deg)
  g_ref[...] = u_ref[...] * dinv
  dinv_ref[...] = jnp.broadcast_to(dinv, dinv_ref.shape)


def _post_block(p0, p1, g_ref, dinv, b_ref, lw_ref, lb_ref, a_ref):
  h = (p0 + p1 + g_ref[...]) * dinv + b_ref[...]
  h = jnp.where(h >= 0, h, a_ref[...] * h)
  mu = jnp.mean(h, axis=-1, keepdims=True)
  var = jnp.mean((h - mu) ** 2, axis=-1, keepdims=True)
  return (h - mu) * lax.rsqrt(var + EPS) * lw_ref[...] + lb_ref[...]


def _b2_body(p0_ref, p1_ref, g1_ref, dv_ref, b_ref, lw_ref, lb_ref,
             a_ref, w_ref, out_ref):
  dinv = dv_ref[...][:, 0:1]
  h = _post_block(p0_ref[0], p1_ref[0], g1_ref, dinv, b_ref, lw_ref, lb_ref,
                  a_ref)
  out_ref[...] = jnp.dot(h, w_ref[...],
                         preferred_element_type=jnp.float32) * dinv


def _b3_body(p0_ref, p1_ref, g2_ref, dv_ref, b_ref, lw_ref, lb_ref,
             a_ref, out_ref):
  dinv = dv_ref[...][:, 0:1]
  out_ref[...] = _post_block(p0_ref[0], p1_ref[0], g2_ref, dinv, b_ref,
                             lw_ref, lb_ref, a_ref)


def _row_spec():
  return pl.BlockSpec((BR, D), lambda i: (i, 0))


def _part_spec(c):
  return pl.BlockSpec((1, BR, D), lambda i, c=c: (c, i, 0))


def _degp_spec(c):
  return pl.BlockSpec((1, BR, DEG_W), lambda i, c=c: (c, i, 0))


def _dinv_spec():
  return pl.BlockSpec((BR, 8), lambda i: (i, 0))


def _vec_spec():
  return pl.BlockSpec((1, D), lambda i: (0, 0))


def _full_spec():
  return pl.BlockSpec((D, D), lambda i: (0, 0))


_GRID = (N_PAD // BR,)
_ROWS_OUT = jax.ShapeDtypeStruct((N_PAD, D), jnp.float32)


def kernel(x, edge_index, W1, b1, W2, b2, prelu_a, ln_w, ln_b):
  e = edge_index.shape[1]
  ept = -(-e // (NW * CHUNK * IBLK)) * CHUNK * IBLK
  e_pad = ept * NW
  nch = ept // CHUNK
  pad = N_NODES + jnp.arange(e_pad - e, dtype=jnp.int32) % (N_PAD - N_NODES)
  nblk = nch // IBLK
  src3 = jnp.concatenate([edge_index[0].astype(jnp.int32), pad])
  src3 = src3.reshape(NW, nblk, IBLK, CHUNK)
  dstf = jnp.concatenate([edge_index[1].astype(jnp.int32), pad])
  dst3 = dstf.reshape(NW, nblk, IBLK, CHUNK)
  dst = dstf.reshape(NW, nch, CHUNK)
  x_pad = jnp.pad(x, ((0, N_PAD - N_NODES), (0, 0)))

  ones_c = jnp.ones((CHUNK, DEG_W), jnp.float32)
  zeros_d = jnp.zeros((ROWS_PT, DEG_W), jnp.float32)
  zeros_r = jnp.zeros((ROWS_PT, D), jnp.float32)

  b1v = b1.reshape(1, D)
  b2v = b2.reshape(1, D)
  lwv = ln_w.reshape(1, D)
  lbv = ln_b.reshape(1, D)
  av = jnp.broadcast_to(prelu_a.reshape(1, 1), (1, D))

  degp = _deg_kernel(nch)(dst, ones_c, zeros_d)

  u1 = pl.pallas_call(
      _mm_body,
      grid=_GRID,
      in_specs=[_row_spec(), _full_spec()],
      out_specs=_row_spec(),
      out_shape=_ROWS_OUT,
  )(x_pad, W1)

  g1, dinv = pl.pallas_call(
      _b1_body,
      grid=_GRID,
      in_specs=[_row_spec(), _degp_spec(0), _degp_spec(1)],
      out_specs=[_row_spec(), _dinv_spec()],
      out_shape=[_ROWS_OUT, jax.ShapeDtypeStruct((N_PAD, 8), jnp.float32)],
  )(u1, degp, degp)

  p = _scatter_kernel(nch)(g1, src3, dst3, zeros_r)

  g2 = pl.pallas_call(
      _b2_body,
      grid=_GRID,
      in_specs=[_part_spec(0), _part_spec(1), _row_spec(), _dinv_spec(),
                _vec_spec(), _vec_spec(), _vec_spec(), _vec_spec(),
                _full_spec()],
      out_specs=_row_spec(),
      out_shape=_ROWS_OUT,
  )(p, p, g1, dinv, b1v, lwv, lbv, av, W2)

  p2 = _scatter_kernel(nch)(g2, src3, dst3, zeros_r)

  out = pl.pallas_call(
      _b3_body,
      grid=(N_NODES // BR3,),
      in_specs=[
          pl.BlockSpec((1, BR3, D), lambda i: (0, i, 0)),
          pl.BlockSpec((1, BR3, D), lambda i: (1, i, 0)),
          pl.BlockSpec((BR3, D), lambda i: (i, 0)),
          pl.BlockSpec((BR3, 8), lambda i: (i, 0)),
          _vec_spec(), _vec_spec(), _vec_spec(), _vec_spec(),
      ],
      out_specs=pl.BlockSpec((BR3, D), lambda i: (i, 0)),
      out_shape=jax.ShapeDtypeStruct((N_NODES, D), jnp.float32),
  )(p2, p2, g2, dinv, b2v, lwv, lbv, av)

  return out
